# Initial kernel scaffold; baseline (speedup 1.0000x reference)
import jax
import jax.numpy as jnp
from jax.experimental import pallas as pl


def kernel(x, W1, b1, W2, b2, W3, b3, W4, b4, w_gate):
    raise NotImplementedError("write your pallas kernel here")


# speedup vs baseline: 1.1427x; 1.1427x over previous
import jax
import jax.numpy as jnp
from jax.experimental import pallas as pl
from jax.experimental.pallas import tpu as pltpu


def _moe_kernel(x_ref, w1_ref, w2_ref, w3_ref, w4_ref, bs_ref, wg_ref, out_ref):
    x = x_ref[:]
    h = w1_ref.shape[1]
    wg = wg_ref[:].astype(jnp.bfloat16)
    zs = []
    logits = None
    for e, w_ref in enumerate((w1_ref, w2_ref, w3_ref, w4_ref)):
        z = jnp.dot(x, w_ref[:], preferred_element_type=jnp.float32)
        z = jnp.maximum(z + bs_ref[e][None, :], 0.0)
        lg = jnp.dot(z.astype(jnp.bfloat16), wg[e * h:(e + 1) * h, :],
                     preferred_element_type=jnp.float32)
        logits = lg if logits is None else logits + lg
        zs.append(z)
    gates = jax.nn.softmax(logits, axis=1)
    acc = gates[:, 0:1] * zs[0]
    for e in range(1, 4):
        acc = acc + gates[:, e:e + 1] * zs[e]
    out_ref[:] = acc


def kernel(x, W1, b1, W2, b2, W3, b3, W4, b4, w_gate):
    n, d_in = x.shape
    h = W1.shape[1]
    xb = x.astype(jnp.bfloat16)
    ws = [W.astype(jnp.bfloat16) for W in (W1, W2, W3, W4)]
    bs = jnp.stack([b1, b2, b3, b4])
    tm = 512
    grid = (n // tm,)
    wspec = pl.BlockSpec((d_in, h), lambda i: (0, 0))
    return pl.pallas_call(
        _moe_kernel,
        grid=grid,
        in_specs=[
            pl.BlockSpec((tm, d_in), lambda i: (i, 0)),
            wspec, wspec, wspec, wspec,
            pl.BlockSpec((4, h), lambda i: (0, 0)),
            pl.BlockSpec((4 * h, 4), lambda i: (0, 0)),
        ],
        out_specs=pl.BlockSpec((tm, h), lambda i: (i, 0)),
        out_shape=jax.ShapeDtypeStruct((n, h), jnp.float32),
        compiler_params=pltpu.CompilerParams(
            dimension_semantics=("arbitrary",),
        ),
    )(xb, *ws, bs, w_gate)


# speedup vs baseline: 1.3258x; 1.1603x over previous
import jax
import jax.numpy as jnp
from jax.experimental import pallas as pl
from jax.experimental.pallas import tpu as pltpu


def _moe_kernel(x_ref, w_ref, b_ref, wg_ref, out_ref):
    h = out_ref.shape[1]
    xb = x_ref[:].astype(jnp.bfloat16)
    zc = jnp.dot(xb, w_ref[:], preferred_element_type=jnp.float32)
    zc = jnp.maximum(zc + b_ref[0][None, :], 0.0)
    logits = jnp.dot(zc.astype(jnp.bfloat16), wg_ref[:].astype(jnp.bfloat16),
                     preferred_element_type=jnp.float32)
    gates = jax.nn.softmax(logits, axis=1)
    acc = gates[:, 0:1] * zc[:, 0:h]
    for e in range(1, 4):
        acc = acc + gates[:, e:e + 1] * zc[:, e * h:(e + 1) * h]
    out_ref[:] = acc


def kernel(x, W1, b1, W2, b2, W3, b3, W4, b4, w_gate):
    n, d_in = x.shape
    h = W1.shape[1]
    wc = jnp.concatenate([W1, W2, W3, W4], axis=1).astype(jnp.bfloat16)
    bc = jnp.concatenate([b1, b2, b3, b4]).reshape(1, 4 * h)
    tm = 256
    grid = (n // tm,)
    return pl.pallas_call(
        _moe_kernel,
        grid=grid,
        in_specs=[
            pl.BlockSpec((tm, d_in), lambda i: (i, 0)),
            pl.BlockSpec((d_in, 4 * h), lambda i: (0, 0)),
            pl.BlockSpec((1, 4 * h), lambda i: (0, 0)),
            pl.BlockSpec((4 * h, 4), lambda i: (0, 0)),
        ],
        out_specs=pl.BlockSpec((tm, h), lambda i: (i, 0)),
        out_shape=jax.ShapeDtypeStruct((n, h), jnp.float32),
        compiler_params=pltpu.CompilerParams(
            dimension_semantics=("arbitrary",),
        ),
    )(x, wc, bc, w_gate)
